# baseline (speedup 1.0000x reference)
import functools

import jax
import jax.numpy as jnp
from jax import lax
from jax.experimental import pallas as pl
from jax.experimental.pallas import tpu as pltpu
from jax.experimental.pallas import tpu_sc as plsc

_NC = 2
_NS = 16
_CH = 128


def _sc_body(nch, stripe, tail, xs_hbm, src_hbm, dst_hbm, w_hbm, out_hbm,
             acc, src_v, dst_v, w_v, r0, r1, r2, g0, g1, g2, s0, s1, s2):
    c = lax.axis_index("c")
    s = lax.axis_index("s")
    rows = (r0, r1, r2)
    sg = (g0, g1, g2)
    ss = (s0, s1, s2)
    dh = r0.shape[1]
    ngrp = dh // 16

    zeros16 = jnp.zeros((16,), jnp.float32)
    for i in range(16):
        for g in range(ngrp):
            r0[i, pl.ds(g * 16, 16)] = zeros16

    def zcp(i, carry):
        pltpu.sync_copy(r0.at[pl.ds(0, 16)],
                        acc.at[pl.ds(s * stripe + i * 16, 16)])
        return carry

    lax.fori_loop(0, stripe // 16, zcp, 0)
    if tail:
        @pl.when(s == 0)
        def _():
            pltpu.sync_copy(r0.at[pl.ds(0, tail)],
                            acc.at[pl.ds(_NS * stripe, tail)])
    plsc.subcore_barrier()

    pltpu.sync_copy(src_hbm.at[c, s], src_v)
    pltpu.sync_copy(dst_hbm.at[s], dst_v)
    pltpu.sync_copy(w_hbm.at[s], w_v)

    def gather(j, b):
        pltpu.async_copy(xs_hbm.at[src_v.at[j]], rows[b], sg[b])

    def wait_gather(b):
        pltpu.make_async_copy(xs_hbm.at[src_v.at[0]], rows[b], sg[b]).wait()

    def scatter(j, b):
        pltpu.async_copy(rows[b], acc.at[dst_v.at[j]], ss[b], add=True)

    def wait_scatter(b):
        pltpu.make_async_copy(rows[b], acc.at[dst_v.at[0]], ss[b]).wait()

    def scale(j, b):
        rb = rows[b]

        def sc16(k16, carry2):
            w16 = w_v[j, pl.ds(k16 * 16, 16)]
            for i in range(16):
                wk = lax.broadcast_in_dim(
                    lax.squeeze(lax.slice(w16, (i,), (i + 1,)), (0,)), (16,), ())
                k = k16 * 16 + i
                for g in range(ngrp):
                    rb[k, pl.ds(g * 16, 16)] = rb[k, pl.ds(g * 16, 16)] * wk
            return carry2

        lax.fori_loop(0, _CH // 16, sc16, 0)

    nt = nch // 3
    gather(0, 0)
    gather(1, 1)

    def triple(jp, carry):
        for b in range(3):
            j = 3 * jp + b
            wait_gather(b)
            scale(j, b)
            scatter(j, b)
            nb = (b + 2) % 3
            if b == 0:
                @pl.when(jp >= 1)
                def _():
                    wait_scatter(nb)
                gather(j + 2, nb)
            elif b == 1:
                wait_scatter(nb)
                gather(j + 2, nb)
            else:
                @pl.when(jp <= nt - 2)
                def _():
                    wait_scatter(nb)
                    gather(j + 2, nb)
        return carry

    lax.fori_loop(0, nt, triple, 0)

    wait_gather(0)
    scale(nch - 1, 0)
    scatter(nch - 1, 0)
    wait_scatter(0)
    wait_scatter(1)
    wait_scatter(2)
    plsc.subcore_barrier()

    pltpu.sync_copy(acc.at[pl.ds(s * stripe, stripe)],
                    out_hbm.at[c, pl.ds(s * stripe, stripe)])
    if tail:
        @pl.when(s == 0)
        def _():
            pltpu.sync_copy(acc.at[pl.ds(_NS * stripe, tail)],
                            out_hbm.at[c, pl.ds(_NS * stripe, tail)])


def _mm_body(dh, p_ref, w_ref, o_ref):
    o_ref[...] = (
        lax.dot_general(p_ref[0], w_ref[:, :dh], (((1,), (1,)), ((), ())),
                        preferred_element_type=jnp.float32)
        + lax.dot_general(p_ref[1], w_ref[:, dh:], (((1,), (1,)), ((), ())),
                          preferred_element_type=jnp.float32))


def kernel(node_emb, edge_index, edge_weight, W):
    n, d = node_emb.shape
    e = edge_index.shape[1]
    dh = d // 2
    assert d == 128 and e % _NS == 0
    ept = e // _NS
    npad = -(-ept // _CH) * _CH
    nch = npad // _CH
    assert nch % 3 == 1 and nch >= 4
    stripe = (n // _NS) // 8 * 8
    tail = n - stripe * _NS
    assert stripe % 16 == 0 and tail <= 16

    src = edge_index[0].astype(jnp.int32).reshape(_NS, ept)
    dst = edge_index[1].astype(jnp.int32).reshape(_NS, ept)
    wv = edge_weight.reshape(_NS, ept)
    if npad != ept:
        pad = ((0, 0), (0, npad - ept))
        src = jnp.pad(src, pad)
        dst = jnp.pad(dst, pad)
        wv = jnp.pad(wv, pad)
    src3 = (src[None] + (jnp.arange(_NC, dtype=jnp.int32) * n)[:, None, None]
            ).reshape(_NC, _NS, nch, _CH)
    dst3 = dst.reshape(_NS, nch, _CH)
    w3 = wv.reshape(_NS, nch, _CH)
    xs = node_emb.reshape(n, _NC, dh).transpose(1, 0, 2).reshape(_NC * n, dh)

    mesh = plsc.VectorSubcoreMesh(core_axis_name="c", subcore_axis_name="s")
    partials = pl.kernel(
        functools.partial(_sc_body, nch, stripe, tail),
        out_type=jax.ShapeDtypeStruct((_NC, n, dh), jnp.float32),
        mesh=mesh,
        compiler_params=pltpu.CompilerParams(use_tc_tiling_on_sc=False),
        scratch_types=[
            pltpu.VMEM_SHARED((n, dh), jnp.float32),
            pltpu.VMEM((nch, _CH), jnp.int32),
            pltpu.VMEM((nch, _CH), jnp.int32),
            pltpu.VMEM((nch, _CH), jnp.float32),
            pltpu.VMEM((_CH, dh), jnp.float32),
            pltpu.VMEM((_CH, dh), jnp.float32),
            pltpu.VMEM((_CH, dh), jnp.float32),
            pltpu.SemaphoreType.DMA,
            pltpu.SemaphoreType.DMA,
            pltpu.SemaphoreType.DMA,
            pltpu.SemaphoreType.DMA,
            pltpu.SemaphoreType.DMA,
            pltpu.SemaphoreType.DMA,
        ],
    )(xs, src3, dst3, w3)

    bn = 1000
    out = pl.pallas_call(
        functools.partial(_mm_body, dh),
        grid=(n // bn,),
        in_specs=[
            pl.BlockSpec((_NC, bn, dh), lambda i: (0, i, 0)),
            pl.BlockSpec((d, d), lambda i: (0, 0)),
        ],
        out_specs=pl.BlockSpec((bn, d), lambda i: (i, 0)),
        out_shape=jax.ShapeDtypeStruct((n, d), jnp.float32),
    )(partials, W)
    return out


# speedup vs baseline: 1.0188x; 1.0188x over previous
import functools

import jax
import jax.numpy as jnp
from jax import lax
from jax.experimental import pallas as pl
from jax.experimental.pallas import tpu as pltpu
from jax.experimental.pallas import tpu_sc as plsc

_NC = 2
_NS = 16
_CH = 128


def _sc_body(nch, stripe, tail, xs_hbm, src_hbm, dst_hbm, w_hbm, out_hbm,
             acc, src_v, dst_v, w0, w1, w2, r0, r1, r2, g0, g1, g2, s0, s1, s2):
    c = lax.axis_index("c")
    s = lax.axis_index("s")
    rows = (r0, r1, r2)
    wexp = (w0, w1, w2)
    sg = (g0, g1, g2)
    ss = (s0, s1, s2)
    dh = r0.shape[1]
    ngrp = dh // 16

    zeros16 = jnp.zeros((16,), jnp.float32)
    for i in range(16):
        for g in range(ngrp):
            r0[i, pl.ds(g * 16, 16)] = zeros16

    def zcp(i, carry):
        pltpu.sync_copy(r0.at[pl.ds(0, 16)],
                        acc.at[pl.ds(s * stripe + i * 16, 16)])
        return carry

    lax.fori_loop(0, stripe // 16, zcp, 0)
    if tail:
        @pl.when(s == 0)
        def _():
            pltpu.sync_copy(r0.at[pl.ds(0, tail)],
                            acc.at[pl.ds(_NS * stripe, tail)])
    plsc.subcore_barrier()

    pltpu.sync_copy(src_hbm.at[c, s], src_v)
    pltpu.sync_copy(dst_hbm.at[s], dst_v)

    def gather(j, b):
        pltpu.async_copy(xs_hbm.at[src_v.at[j]], rows[b], sg[b])
        pltpu.async_copy(w_hbm.at[s, j], wexp[b], sg[b])

    def wait_gather(b):
        pltpu.make_async_copy(xs_hbm.at[src_v.at[0]], rows[b], sg[b]).wait()
        pltpu.make_async_copy(w_hbm.at[s, 0], wexp[b], sg[b]).wait()

    def scatter(j, b):
        pltpu.async_copy(rows[b], acc.at[dst_v.at[j]], ss[b], add=True)

    def wait_scatter(b):
        pltpu.make_async_copy(rows[b], acc.at[dst_v.at[0]], ss[b]).wait()

    def scale(j, b):
        rb = rows[b]
        wb = wexp[b]

        def sc16(k16, carry2):
            base = k16 * 16
            for i in range(16):
                k = base + i
                wk = wb[pl.ds(k * 16, 16)]
                for g in range(ngrp):
                    rb[k, pl.ds(g * 16, 16)] = rb[k, pl.ds(g * 16, 16)] * wk
            return carry2

        lax.fori_loop(0, _CH // 16, sc16, 0)

    nt = nch // 3
    gather(0, 0)
    gather(1, 1)

    def triple(jp, carry):
        for b in range(3):
            j = 3 * jp + b
            wait_gather(b)
            scale(j, b)
            scatter(j, b)
            nb = (b + 2) % 3
            if b == 0:
                @pl.when(jp >= 1)
                def _():
                    wait_scatter(nb)
                gather(j + 2, nb)
            elif b == 1:
                wait_scatter(nb)
                gather(j + 2, nb)
            else:
                @pl.when(jp <= nt - 2)
                def _():
                    wait_scatter(nb)
                    gather(j + 2, nb)
        return carry

    lax.fori_loop(0, nt, triple, 0)

    wait_gather(0)
    scale(nch - 1, 0)
    scatter(nch - 1, 0)
    wait_scatter(0)
    wait_scatter(1)
    wait_scatter(2)
    plsc.subcore_barrier()

    pltpu.sync_copy(acc.at[pl.ds(s * stripe, stripe)],
                    out_hbm.at[c, pl.ds(s * stripe, stripe)])
    if tail:
        @pl.when(s == 0)
        def _():
            pltpu.sync_copy(acc.at[pl.ds(_NS * stripe, tail)],
                            out_hbm.at[c, pl.ds(_NS * stripe, tail)])


def _mm_body(dh, p_ref, w_ref, o_ref):
    o_ref[...] = (
        lax.dot_general(p_ref[0], w_ref[:, :dh], (((1,), (1,)), ((), ())),
                        preferred_element_type=jnp.float32)
        + lax.dot_general(p_ref[1], w_ref[:, dh:], (((1,), (1,)), ((), ())),
                          preferred_element_type=jnp.float32))


def kernel(node_emb, edge_index, edge_weight, W):
    n, d = node_emb.shape
    e = edge_index.shape[1]
    dh = d // 2
    assert d == 128 and e % _NS == 0
    ept = e // _NS
    npad = -(-ept // _CH) * _CH
    nch = npad // _CH
    assert nch % 3 == 1 and nch >= 4
    stripe = (n // _NS) // 8 * 8
    tail = n - stripe * _NS
    assert stripe % 16 == 0 and tail <= 16

    src = edge_index[0].astype(jnp.int32).reshape(_NS, ept)
    dst = edge_index[1].astype(jnp.int32).reshape(_NS, ept)
    wv = edge_weight.reshape(_NS, ept)
    if npad != ept:
        pad = ((0, 0), (0, npad - ept))
        src = jnp.pad(src, pad)
        dst = jnp.pad(dst, pad)
        wv = jnp.pad(wv, pad)
    src3 = (src[None] + (jnp.arange(_NC, dtype=jnp.int32) * n)[:, None, None]
            ).reshape(_NC, _NS, nch, _CH)
    dst3 = dst.reshape(_NS, nch, _CH)
    wexp = jnp.broadcast_to(wv[:, :, None], (_NS, npad, 16)
                            ).reshape(_NS, nch, _CH * 16)
    xs = node_emb.reshape(n, _NC, dh).transpose(1, 0, 2).reshape(_NC * n, dh)

    mesh = plsc.VectorSubcoreMesh(core_axis_name="c", subcore_axis_name="s")
    partials = pl.kernel(
        functools.partial(_sc_body, nch, stripe, tail),
        out_type=jax.ShapeDtypeStruct((_NC, n, dh), jnp.float32),
        mesh=mesh,
        compiler_params=pltpu.CompilerParams(use_tc_tiling_on_sc=False),
        scratch_types=[
            pltpu.VMEM_SHARED((n, dh), jnp.float32),
            pltpu.VMEM((nch, _CH), jnp.int32),
            pltpu.VMEM((nch, _CH), jnp.int32),
            pltpu.VMEM((_CH * 16,), jnp.float32),
            pltpu.VMEM((_CH * 16,), jnp.float32),
            pltpu.VMEM((_CH * 16,), jnp.float32),
            pltpu.VMEM((_CH, dh), jnp.float32),
            pltpu.VMEM((_CH, dh), jnp.float32),
            pltpu.VMEM((_CH, dh), jnp.float32),
            pltpu.SemaphoreType.DMA,
            pltpu.SemaphoreType.DMA,
            pltpu.SemaphoreType.DMA,
            pltpu.SemaphoreType.DMA,
            pltpu.SemaphoreType.DMA,
            pltpu.SemaphoreType.DMA,
        ],
    )(xs, src3, dst3, wexp)

    bn = 1000
    out = pl.pallas_call(
        functools.partial(_mm_body, dh),
        grid=(n // bn,),
        in_specs=[
            pl.BlockSpec((_NC, bn, dh), lambda i: (0, i, 0)),
            pl.BlockSpec((d, d), lambda i: (0, 0)),
        ],
        out_specs=pl.BlockSpec((bn, d), lambda i: (i, 0)),
        out_shape=jax.ShapeDtypeStruct((n, d), jnp.float32),
    )(partials, W)
    return out
